# Initial kernel scaffold; baseline (speedup 1.0000x reference)
import jax
import jax.numpy as jnp
from jax.experimental import pallas as pl


def kernel(logits_exp, logits_ama):
    raise NotImplementedError("write your pallas kernel here")


# speedup vs baseline: 19.9978x; 19.9978x over previous
import functools
import math

import jax
import jax.numpy as jnp
from jax import lax
from jax.experimental import pallas as pl
from jax.experimental.pallas import tpu as pltpu
from jax.experimental.pallas import tpu_sc as plsc

ALPHA = 0.9
K = 10
LANES = 16
GROUP = 8
NC, NS = 2, 16
NW = NC * NS
NEG_INF = float("-inf")
INT_MIN = -2**31
XOR_MASK = 0x7FFFFFFF
LN_EPS = math.log(1e-8)


def _sortable_key(x):
    b = lax.bitcast_convert_type(x, jnp.int32)
    return jnp.where(b >= 0, b, b ^ XOR_MASK)


def _key_to_float(k):
    b = jnp.where(k >= 0, k, k ^ XOR_MASK)
    return lax.bitcast_convert_type(b, jnp.float32)


def _row_stats(keys, x, valid, ksel):
    g = keys.shape[0]

    def bs_body(i, cur):
        bit = jnp.left_shift(jnp.int32(1), 31 - i)
        cand = cur + bit
        cnt = jnp.sum((keys >= cand).astype(jnp.int32), axis=1, keepdims=True)
        return jnp.where(cnt >= ksel, cand, cur)

    tkey = lax.fori_loop(0, 32, bs_body, jnp.full((g, 1), INT_MIN, jnp.int32))

    rmax = jnp.max(jnp.where(valid, x, NEG_INF), axis=1, keepdims=True)
    ex = jnp.where(keys > tkey, jnp.exp(x - rmax), 0.0)
    s_gt = jnp.sum(ex, axis=1, keepdims=True)
    n_gt = jnp.sum((keys > tkey).astype(jnp.int32), axis=1, keepdims=True)
    tval = _key_to_float(tkey)
    n_tie = (ksel - n_gt).astype(jnp.float32)
    z = s_gt + n_tie * jnp.exp(tval - rmax)
    return tkey, rmax, z


def _tc_body(ksel, v, exp_ref, ama_ref, out_ref, gidx_ref, a_ref, b_ref,
             t_ref, u_ref):
    xe = exp_ref[...]
    xa = ama_ref[...]
    col = lax.broadcasted_iota(jnp.int32, xe.shape, 1)
    valid = col < v

    ke = jnp.where(valid, _sortable_key(xe), INT_MIN)
    ka = jnp.where(valid, _sortable_key(xa), INT_MIN)

    tke, me, ze = _row_stats(ke, xe, valid, ksel)
    tka, ma, za = _row_stats(ka, xa, valid, ksel)
    lze = jnp.log(ze)
    lza = jnp.log(za)

    work = ke
    idxs, pvals, lpes = [], [], []
    for _ in range(K):
        mx = jnp.max(work, axis=1, keepdims=True)
        idx = jnp.min(jnp.where(work == mx, col, jnp.int32(2**31 - 1)),
                      axis=1, keepdims=True)
        work = jnp.where(col == idx, INT_MIN, work)
        vj = _key_to_float(mx)
        idxs.append(idx)
        pvals.append(jnp.exp(vj - me) / ze)
        lpes.append(vj - me - lze)

    p10 = pvals[-1]
    lanes = [9 if j >= K else j for j in range(LANES)]
    conds = [pvals[j] >= ALPHA * p10 for j in range(K)]
    a_cols, b_cols = [], []
    for j in lanes:
        a_cols.append(jnp.where(conds[j], lpes[j] + ma + lza, NEG_INF))
        b_cols.append(jnp.where(conds[j], lpes[j] - LN_EPS, NEG_INF))

    rows = (pl.program_id(0) * GROUP
            + lax.broadcasted_iota(jnp.int32, (xe.shape[0], 1), 0))
    gidx_cols = [rows * v + idxs[j] for j in lanes]

    gidx_ref[...] = jnp.concatenate(gidx_cols, axis=1)
    a_ref[...] = jnp.concatenate(a_cols, axis=1)
    b_ref[...] = jnp.concatenate(b_cols, axis=1)
    t_ref[...] = jnp.broadcast_to(_key_to_float(tka), (xe.shape[0], LANES))
    u_ref[...] = jnp.broadcast_to(ma + lza + LN_EPS, (xe.shape[0], LANES))
    out_ref[...] = jnp.full(xe.shape, NEG_INF, jnp.float32)


def _tc_call(logits_exp, logits_ama, ksel):
    r, v = logits_exp.shape
    grid = r // GROUP
    small = pl.BlockSpec((GROUP, LANES), lambda i: (i, 0))
    return pl.pallas_call(
        functools.partial(_tc_body, ksel, v),
        grid=(grid,),
        in_specs=[
            pl.BlockSpec((GROUP, v), lambda i: (i, 0)),
            pl.BlockSpec((GROUP, v), lambda i: (i, 0)),
        ],
        out_specs=[pl.BlockSpec((GROUP, v), lambda i: (i, 0)),
                   small, small, small, small, small],
        out_shape=[
            jax.ShapeDtypeStruct((r, v), jnp.float32),
            jax.ShapeDtypeStruct((r, LANES), jnp.int32),
            jax.ShapeDtypeStruct((r, LANES), jnp.float32),
            jax.ShapeDtypeStruct((r, LANES), jnp.float32),
            jax.ShapeDtypeStruct((r, LANES), jnp.float32),
            jax.ShapeDtypeStruct((r, LANES), jnp.float32),
        ],
    )(logits_exp, logits_ama)


def _sc_body(n_per_w, out_ref, gidx_hbm, a_hbm, b_hbm, t_hbm, u_hbm, ama_hbm,
             gidx_v, a_v, b_v, t_v, u_v, xa_v, sc_v, sem):
    wid = lax.axis_index("s") * NC + lax.axis_index("c")
    base = wid * n_per_w
    cps = [
        pltpu.async_copy(gidx_hbm.at[pl.ds(base, n_per_w)], gidx_v, sem),
        pltpu.async_copy(a_hbm.at[pl.ds(base, n_per_w)], a_v, sem),
        pltpu.async_copy(b_hbm.at[pl.ds(base, n_per_w)], b_v, sem),
        pltpu.async_copy(t_hbm.at[pl.ds(base, n_per_w)], t_v, sem),
        pltpu.async_copy(u_hbm.at[pl.ds(base, n_per_w)], u_v, sem),
    ]
    for cp in cps:
        cp.wait()
    pltpu.async_copy(ama_hbm.at[gidx_v], xa_v, sem).wait()
    for i in range(n_per_w // LANES):
        sl = pl.ds(i * LANES, LANES)
        xa = xa_v[sl]
        kept = xa >= t_v[sl]
        val = a_v[sl] - xa - jnp.exp(u_v[sl] - xa)
        sc_v[sl] = jnp.where(kept, val, b_v[sl])
    pltpu.async_copy(sc_v, out_ref.at[gidx_v], sem).wait()


def kernel(logits_exp, logits_ama):
    r, v = logits_exp.shape
    ksel = math.ceil((1.0 - ALPHA) * v)
    out, gidx, a, b, t, u = _tc_call(logits_exp, logits_ama, ksel)

    n_per_w = (r * LANES) // NW
    mesh = plsc.VectorSubcoreMesh(core_axis_name="c", subcore_axis_name="s",
                                  num_cores=NC, num_subcores=NS)
    sc_scatter = functools.partial(
        pl.kernel,
        out_type=(),
        mesh=mesh,
        scratch_types=[
            pltpu.VMEM((n_per_w,), jnp.int32),
            pltpu.VMEM((n_per_w,), jnp.float32),
            pltpu.VMEM((n_per_w,), jnp.float32),
            pltpu.VMEM((n_per_w,), jnp.float32),
            pltpu.VMEM((n_per_w,), jnp.float32),
            pltpu.VMEM((n_per_w,), jnp.float32),
            pltpu.VMEM((n_per_w,), jnp.float32),
            pltpu.SemaphoreType.DMA,
        ],
    )(functools.partial(_sc_body, n_per_w))

    out_ref = jax.new_ref(out.reshape(r * v))
    sc_scatter(out_ref, gidx.reshape(-1), a.reshape(-1), b.reshape(-1),
               t.reshape(-1), u.reshape(-1), logits_ama.reshape(-1))
    return out_ref[...].reshape(r, v)


# speedup vs baseline: 26.6595x; 1.3331x over previous
import functools
import math

import jax
import jax.numpy as jnp
from jax import lax
from jax.experimental import pallas as pl
from jax.experimental.pallas import tpu as pltpu
from jax.experimental.pallas import tpu_sc as plsc

ALPHA = 0.9
K = 10
NSLOT = 12
LANES = 16
GROUP = 8
NC, NS = 2, 16
NW = NC * NS
NEG_INF = float("-inf")
INT_MIN = -2**31
BIG = 2**31 - 1
XOR_MASK = 0x7FFFFFFF
LN_EPS = math.log(1e-8)


def _sortable_key(x):
    b = lax.bitcast_convert_type(x, jnp.int32)
    return jnp.where(b >= 0, b, b ^ XOR_MASK)


def _key_to_float(k):
    b = jnp.where(k >= 0, k, k ^ XOR_MASK)
    return lax.bitcast_convert_type(b, jnp.float32)


def _count_ge(keys, cand):
    return jnp.sum((keys >= cand).astype(jnp.int32), axis=1, keepdims=True)


def _logsumexp_kept(keys, x, tkey, rmax, ksel):
    ex = jnp.where(keys > tkey, jnp.exp(x - rmax), 0.0)
    s_gt = jnp.sum(ex, axis=1, keepdims=True)
    n_gt = jnp.sum((keys > tkey).astype(jnp.int32), axis=1, keepdims=True)
    n_tie = (ksel - n_gt).astype(jnp.float32)
    z = s_gt + n_tie * jnp.exp(_key_to_float(tkey) - rmax)
    return z


def _tc_body(ksel, v, exp_ref, ama_ref, out_ref, gidx_ref, me_ref, t10_ref,
             d_ref, u_ref, e_ref, ta_ref):
    xe = exp_ref[...]
    xa = ama_ref[...]
    g = xe.shape[0]
    col = lax.broadcasted_iota(jnp.int32, xe.shape, 1)
    valid = col < v

    ke = jnp.where(valid, _sortable_key(xe), INT_MIN)
    ka = jnp.where(valid, _sortable_key(xa), INT_MIN)

    def bs_body(i, carry):
        cek, ce10, cak = carry
        bit = jnp.left_shift(jnp.int32(1), 31 - i)
        nek, ne10, nak = cek + bit, ce10 + bit, cak + bit
        cek = jnp.where(_count_ge(ke, nek) >= ksel, nek, cek)
        ce10 = jnp.where(_count_ge(ke, ne10) >= K, ne10, ce10)
        cak = jnp.where(_count_ge(ka, nak) >= ksel, nak, cak)
        return cek, ce10, cak

    init = jnp.full((g, 1), INT_MIN, jnp.int32)
    tke, t10k, tka = lax.fori_loop(0, 32, bs_body, (init, init, init))

    me = jnp.max(jnp.where(valid, xe, NEG_INF), axis=1, keepdims=True)
    ma = jnp.max(jnp.where(valid, xa, NEG_INF), axis=1, keepdims=True)
    ze = _logsumexp_kept(ke, xe, tke, me, ksel)
    za = _logsumexp_kept(ka, xa, tka, ma, ksel)
    lze = jnp.log(ze)
    lza = jnp.log(za)

    strictcol = jnp.where(ke > t10k, col, BIG)
    tiedcol = jnp.where(ke == t10k, col, BIG)
    n_strict = jnp.sum((ke > t10k).astype(jnp.int32), axis=1, keepdims=True)
    quota = K - n_strict

    def extract(cols, n):
        out = []
        prev = jnp.full((g, 1), -1, jnp.int32)
        for _ in range(n):
            nxt = jnp.min(jnp.where(cols > prev, cols, BIG), axis=1,
                          keepdims=True)
            out.append(nxt)
            prev = nxt
        return out

    s_slots = extract(strictcol, K - 1)
    t_slots = extract(tiedcol, NSLOT - (K - 1))

    mn = jnp.min(ke, axis=1, keepdims=True)
    safe = jnp.min(jnp.where(ke == mn, col, BIG), axis=1, keepdims=True)

    rows = (pl.program_id(0) * GROUP
            + lax.broadcasted_iota(jnp.int32, (g, 1), 0))
    ineligible_enc = -(rows * v + safe) - 1
    gcols = []
    for s in range(LANES):
        if s < K - 1:
            c = s_slots[s]
            ok = c != BIG
        elif s < NSLOT:
            c = t_slots[s - (K - 1)]
            ok = (c != BIG) & (s - (K - 1) < quota)
        else:
            c = None
            ok = None
        if c is None:
            enc = ineligible_enc
        else:
            enc = jnp.where(ok, rows * v + jnp.minimum(c, v - 1),
                            ineligible_enc)
        gcols.append(enc)
    gidx_ref[...] = jnp.concatenate(gcols, axis=1)

    shape16 = (g, LANES)
    me_ref[...] = jnp.broadcast_to(me, shape16)
    t10_ref[...] = jnp.broadcast_to(_key_to_float(t10k), shape16)
    d_ref[...] = jnp.broadcast_to(me + lze - ma - lza, shape16)
    u_ref[...] = jnp.broadcast_to(ma + lza + LN_EPS, shape16)
    e_ref[...] = jnp.broadcast_to(me + lze + LN_EPS, shape16)
    ta_ref[...] = jnp.broadcast_to(_key_to_float(tka), shape16)
    out_ref[...] = jnp.full(xe.shape, NEG_INF, jnp.float32)


def _tc_call(logits_exp, logits_ama, ksel):
    r, v = logits_exp.shape
    grid = r // GROUP
    small = pl.BlockSpec((GROUP, LANES), lambda i: (i, 0))
    sds = jax.ShapeDtypeStruct
    return pl.pallas_call(
        functools.partial(_tc_body, ksel, v),
        grid=(grid,),
        in_specs=[
            pl.BlockSpec((GROUP, v), lambda i: (i, 0)),
            pl.BlockSpec((GROUP, v), lambda i: (i, 0)),
        ],
        out_specs=[pl.BlockSpec((GROUP, v), lambda i: (i, 0))] + [small] * 7,
        out_shape=[sds((r, v), jnp.float32), sds((r, LANES), jnp.int32)]
                  + [sds((r, LANES), jnp.float32)] * 6,
    )(logits_exp, logits_ama)


def _sc_body(n_per_w, out_ref, gidx_hbm, me_hbm, t10_hbm, d_hbm, u_hbm,
             e_hbm, ta_hbm, exp_hbm, ama_hbm,
             gidx_v, me_v, t10_v, d_v, u_v, e_v, ta_v, xe_v, xa_v, sc_v, sem):
    wid = lax.axis_index("s") * NC + lax.axis_index("c")
    base = wid * n_per_w
    sl_all = pl.ds(base, n_per_w)
    cps = [
        pltpu.async_copy(gidx_hbm.at[sl_all], gidx_v, sem),
        pltpu.async_copy(me_hbm.at[sl_all], me_v, sem),
        pltpu.async_copy(t10_hbm.at[sl_all], t10_v, sem),
        pltpu.async_copy(d_hbm.at[sl_all], d_v, sem),
        pltpu.async_copy(u_hbm.at[sl_all], u_v, sem),
        pltpu.async_copy(e_hbm.at[sl_all], e_v, sem),
        pltpu.async_copy(ta_hbm.at[sl_all], ta_v, sem),
    ]
    for cp in cps:
        cp.wait()
    for i in range(n_per_w // LANES):
        sl = pl.ds(i * LANES, LANES)
        enc = gidx_v[sl]
        gidx_v[sl] = jnp.where(enc < 0, -enc - 1, enc)
        sc_v[sl] = jnp.where(enc < 0, 0.0, 1.0)
    pltpu.async_copy(exp_hbm.at[gidx_v], xe_v, sem).wait()
    pltpu.async_copy(ama_hbm.at[gidx_v], xa_v, sem).wait()
    for i in range(n_per_w // LANES):
        sl = pl.ds(i * LANES, LANES)
        xe = xe_v[sl]
        xa = xa_v[sl]
        t10 = t10_v[sl]
        me = me_v[sl]
        eligible = sc_v[sl] > 0.5
        cond = jnp.exp(xe - me) >= ALPHA * jnp.exp(t10 - me)
        kept_ama = xa >= ta_v[sl]
        val = jnp.where(kept_ama,
                        xe - xa - d_v[sl] - jnp.exp(u_v[sl] - xa),
                        xe - e_v[sl])
        sc_v[sl] = jnp.where(eligible & cond, val, NEG_INF)
    pltpu.async_copy(sc_v, out_ref.at[gidx_v], sem).wait()


def kernel(logits_exp, logits_ama):
    r, v = logits_exp.shape
    ksel = math.ceil((1.0 - ALPHA) * v)
    out, gidx, me, t10, d, u, e, ta = _tc_call(logits_exp, logits_ama, ksel)

    n_per_w = (r * LANES) // NW
    mesh = plsc.VectorSubcoreMesh(core_axis_name="c", subcore_axis_name="s",
                                  num_cores=NC, num_subcores=NS)
    f32, i32 = jnp.float32, jnp.int32
    sc_scatter = functools.partial(
        pl.kernel,
        out_type=(),
        mesh=mesh,
        scratch_types=[pltpu.VMEM((n_per_w,), i32)]
                      + [pltpu.VMEM((n_per_w,), f32)] * 9
                      + [pltpu.SemaphoreType.DMA],
    )(functools.partial(_sc_body, n_per_w))

    out_ref = jax.new_ref(out.reshape(r * v))
    sc_scatter(out_ref, gidx.reshape(-1), me.reshape(-1), t10.reshape(-1),
               d.reshape(-1), u.reshape(-1), e.reshape(-1), ta.reshape(-1),
               logits_exp.reshape(-1), logits_ama.reshape(-1))
    return out_ref[...].reshape(r, v)
